# baseline (speedup 1.0000x reference)
import jax
import jax.numpy as jnp
from jax import lax
from jax.experimental import pallas as pl
from jax.experimental.pallas import tpu as pltpu
from jax.experimental.pallas import tpu_sc as plsc

B = 16384
TD = 32
NNUM = 8
VD = 128
QED = 32

NUM_TABLES = 5
NW = 32
BPW = B // NW
HALF = BPW // 2


def _gather_kernel(ta, tb, tc, td, tq,
                   ia, ib, ic, id_, iq,
                   ja, jb, jc, jd, jq,
                   oa, ob, oc, od, oq,
                   idx_v, idx4_v, gbuf, ebuf, gsem):
  wid = lax.axis_index("s") * 2 + lax.axis_index("c")
  base = wid * BPW

  tables = (ta, tb, tc, td, tq)
  idxs = (ia, ib, ic, id_, iq)
  idx4s = (ja, jb, jc, jd, jq)
  outs = (oa, ob, oc, od, oq)

  for f in range(NUM_TABLES):
    pltpu.sync_copy(idxs[f].at[pl.ds(base, BPW)], idx_v)
    pltpu.sync_copy(idx4s[f].at[pl.ds(base, BPW)], idx4_v)
    table = tables[f]

    for c in range(2):
      pltpu.async_copy(
          table.at[idx4_v.at[pl.ds(c * HALF, HALF)]], gbuf, gsem).wait()

      @pl.loop(0, HALF, step=16)
      def _extract(i):
        v = idx_v[pl.ds(c * HALF + i, 16)]
        for j in range(16):
          off = (v[j] & 3) * TD
          row = gbuf.at[i + j]
          erow = ebuf.at[c * HALF + i + j]
          erow[pl.ds(0, 16)] = row[pl.ds(off, 16)]
          erow[pl.ds(16, 16)] = row[pl.ds(off + 16, 16)]

    pltpu.sync_copy(ebuf, outs[f].at[pl.ds(base, BPW)])


def _sc_gather(packed, ids, ids4):
  fn = pl.kernel(
      _gather_kernel,
      out_type=tuple(
          jax.ShapeDtypeStruct((B, TD), jnp.float32)
          for _ in range(NUM_TABLES)),
      mesh=plsc.VectorSubcoreMesh(core_axis_name="c", subcore_axis_name="s"),
      scratch_types=[
          pltpu.VMEM((BPW,), jnp.int32),
          pltpu.VMEM((BPW,), jnp.int32),
          pltpu.VMEM((HALF, 128), jnp.float32),
          pltpu.VMEM((BPW, TD), jnp.float32),
          pltpu.SemaphoreType.DMA,
      ],
  )
  return fn(*packed, *ids, *ids4)


def _mlp_kernel(ea, eb, ec, ed, eq, num, vec,
                nw1, nb1, nw2, nb2, vw, vb, mw1, mb1, mw2, mb2,
                out):
  h = jnp.maximum(
      jnp.dot(num[...], nw1[...], preferred_element_type=jnp.float32)
      + nb1[...], 0.0)
  h = jnp.dot(h, nw2[...], preferred_element_type=jnp.float32) + nb2[...]
  v = jnp.dot(vec[...], vw[...], preferred_element_type=jnp.float32) + vb[...]
  feat = jnp.concatenate(
      [ea[...], eb[...], ec[...], ed[...], eq[...], h, v], axis=1)
  g = jnp.maximum(
      jnp.dot(feat, mw1[...], preferred_element_type=jnp.float32) + mb1[...],
      0.0)
  out[...] = (
      jnp.dot(g, mw2[...], preferred_element_type=jnp.float32) + mb2[...])


def _tc_mlp(emb_a, emb_b, emb_c, emb_d, emb_q, numericals, vec_emb,
            num_W1, num_b1, num_W2, num_b2, vec_W, vec_b,
            merge_W1, merge_b1, merge_W2, merge_b2):
  BB = 2048
  grid = (B // BB,)

  def batch_spec(width):
    return pl.BlockSpec((BB, width), lambda i: (i, 0))

  def full_spec(shape):
    return pl.BlockSpec(shape, lambda i: tuple(0 for _ in shape))

  return pl.pallas_call(
      _mlp_kernel,
      grid=grid,
      in_specs=[
          batch_spec(TD), batch_spec(TD), batch_spec(TD), batch_spec(TD),
          batch_spec(TD), batch_spec(NNUM), batch_spec(VD),
          full_spec(num_W1.shape), full_spec(num_b1.shape),
          full_spec(num_W2.shape), full_spec(num_b2.shape),
          full_spec(vec_W.shape), full_spec(vec_b.shape),
          full_spec(merge_W1.shape), full_spec(merge_b1.shape),
          full_spec(merge_W2.shape), full_spec(merge_b2.shape),
      ],
      out_specs=batch_spec(QED),
      out_shape=jax.ShapeDtypeStruct((B, QED), jnp.float32),
  )(emb_a, emb_b, emb_c, emb_d, emb_q, numericals, vec_emb,
    num_W1, num_b1, num_W2, num_b2, vec_W, vec_b,
    merge_W1, merge_b1, merge_W2, merge_b2)


def kernel(query_id, cat_a, cat_b, cat_c, cat_d, numericals, vec_emb,
           query_table, ct_a, ct_b, ct_c, ct_d,
           num_W1, num_b1, num_W2, num_b2,
           vec_W, vec_b,
           merge_W1, merge_b1, merge_W2, merge_b2):
  ids = [x.astype(jnp.int32)
         for x in (cat_a, cat_b, cat_c, cat_d, query_id)]
  ids4 = [x >> 2 for x in ids]
  packed = [t.reshape(t.shape[0] // 4, 128)
            for t in (ct_a, ct_b, ct_c, ct_d, query_table)]

  ea, eb, ec, ed, eq = _sc_gather(packed, ids, ids4)

  return _tc_mlp(
      ea, eb, ec, ed, eq, numericals, vec_emb,
      num_W1, num_b1.reshape(1, -1), num_W2, num_b2.reshape(1, -1),
      vec_W, vec_b.reshape(1, -1),
      merge_W1, merge_b1.reshape(1, -1), merge_W2, merge_b2.reshape(1, -1))


# speedup vs baseline: 1.4461x; 1.4461x over previous
import jax
import jax.numpy as jnp
from jax import lax
from jax.experimental import pallas as pl
from jax.experimental.pallas import tpu as pltpu
from jax.experimental.pallas import tpu_sc as plsc

B = 16384
TD = 32
NNUM = 8
VD = 128
QED = 32

NUM_TABLES = 5
NW = 32
BPW = B // NW


def _gather_kernel(qt, ca_t, cb_t, cc_t, cd_t,
                   qid, ca, cb, cc, cd,
                   out_q, out_a, out_b, out_c, out_d,
                   idx_hv, rows_v, osem, *gsems):
  nc = 2
  wid = lax.axis_index("s") * nc + lax.axis_index("c")
  base = wid * BPW

  tables = (qt, ca_t, cb_t, cc_t, cd_t)
  idxs = (qid, ca, cb, cc, cd)
  outs = (out_q, out_a, out_b, out_c, out_d)

  for f in range(NUM_TABLES):
    pltpu.sync_copy(idxs[f].at[pl.ds(base, BPW)], idx_hv)
    table = tables[f]

    @pl.loop(0, BPW, step=16)
    def _rows(i):
      v = idx_hv[pl.ds(i, 16)]
      for j in range(16):
        pltpu.async_copy(
            table.at[pl.ds(v[j], 1)], rows_v.at[pl.ds(i + j, 1)], gsems[0])

    pltpu.make_async_copy(table.at[pl.ds(0, BPW)], rows_v, gsems[0]).wait()
    pltpu.sync_copy(rows_v, outs[f].at[pl.ds(base, BPW)])


def _sc_gather(qt, ca_t, cb_t, cc_t, cd_t, qid, ca, cb, cc, cd):
  mesh = plsc.VectorSubcoreMesh(core_axis_name="c", subcore_axis_name="s")
  out_t = tuple(
      jax.ShapeDtypeStruct((B, TD), jnp.float32) for _ in range(NUM_TABLES))
  fn = pl.kernel(
      _gather_kernel,
      out_type=out_t,
      mesh=mesh,
      scratch_types=(
          [pltpu.VMEM((BPW,), jnp.int32),
           pltpu.VMEM((BPW, TD), jnp.float32)]
          + [pltpu.SemaphoreType.DMA for _ in range(2)]),
  )
  return fn(qt, ca_t, cb_t, cc_t, cd_t, qid, ca, cb, cc, cd)


def _mlp_kernel(ea, eb, ec, ed, eq, num, vec,
                nw1, nb1, nw2, nb2, vw, vb, mw1, mb1, mw2, mb2,
                out):
  h = jnp.maximum(
      jnp.dot(num[...], nw1[...], preferred_element_type=jnp.float32)
      + nb1[...], 0.0)
  h = jnp.dot(h, nw2[...], preferred_element_type=jnp.float32) + nb2[...]
  v = jnp.dot(vec[...], vw[...], preferred_element_type=jnp.float32) + vb[...]
  feat = jnp.concatenate(
      [ea[...], eb[...], ec[...], ed[...], eq[...], h, v], axis=1)
  g = jnp.maximum(
      jnp.dot(feat, mw1[...], preferred_element_type=jnp.float32) + mb1[...],
      0.0)
  out[...] = (
      jnp.dot(g, mw2[...], preferred_element_type=jnp.float32) + mb2[...])


def _tc_mlp(emb_a, emb_b, emb_c, emb_d, emb_q, numericals, vec_emb,
            num_W1, num_b1, num_W2, num_b2, vec_W, vec_b,
            merge_W1, merge_b1, merge_W2, merge_b2):
  BB = 2048
  grid = (B // BB,)

  def batch_spec(width):
    return pl.BlockSpec((BB, width), lambda i: (i, 0))

  def full_spec(shape):
    return pl.BlockSpec(shape, lambda i: tuple(0 for _ in shape))

  return pl.pallas_call(
      _mlp_kernel,
      grid=grid,
      in_specs=[
          batch_spec(TD), batch_spec(TD), batch_spec(TD), batch_spec(TD),
          batch_spec(TD), batch_spec(NNUM), batch_spec(VD),
          full_spec(num_W1.shape), full_spec(num_b1.shape),
          full_spec(num_W2.shape), full_spec(num_b2.shape),
          full_spec(vec_W.shape), full_spec(vec_b.shape),
          full_spec(merge_W1.shape), full_spec(merge_b1.shape),
          full_spec(merge_W2.shape), full_spec(merge_b2.shape),
      ],
      out_specs=batch_spec(QED),
      out_shape=jax.ShapeDtypeStruct((B, QED), jnp.float32),
  )(emb_a, emb_b, emb_c, emb_d, emb_q, numericals, vec_emb,
    num_W1, num_b1, num_W2, num_b2, vec_W, vec_b,
    merge_W1, merge_b1, merge_W2, merge_b2)


def kernel(query_id, cat_a, cat_b, cat_c, cat_d, numericals, vec_emb,
           query_table, ct_a, ct_b, ct_c, ct_d,
           num_W1, num_b1, num_W2, num_b2,
           vec_W, vec_b,
           merge_W1, merge_b1, merge_W2, merge_b2):
  qid = query_id.astype(jnp.int32)
  ca = cat_a.astype(jnp.int32)
  cb = cat_b.astype(jnp.int32)
  cc = cat_c.astype(jnp.int32)
  cd = cat_d.astype(jnp.int32)

  eq, ea, eb, ec, ed = _sc_gather(
      query_table, ct_a, ct_b, ct_c, ct_d, qid, ca, cb, cc, cd)

  return _tc_mlp(
      ea, eb, ec, ed, eq, numericals, vec_emb,
      num_W1, num_b1.reshape(1, -1), num_W2, num_b2.reshape(1, -1),
      vec_W, vec_b.reshape(1, -1),
      merge_W1, merge_b1.reshape(1, -1), merge_W2, merge_b2.reshape(1, -1))


# speedup vs baseline: 1.4826x; 1.0252x over previous
import jax
import jax.numpy as jnp
from jax import lax
from jax.experimental import pallas as pl
from jax.experimental.pallas import tpu as pltpu
from jax.experimental.pallas import tpu_sc as plsc

B = 16384
TD = 32
NNUM = 8
VD = 128
QED = 32

NUM_TABLES = 5
NW = 32
BPW = B // NW


def _gather_kernel(*args):
  nc = 2
  wid = lax.axis_index("s") * nc + lax.axis_index("c")
  base = wid * BPW

  nt = (len(args) - 4) // 3
  tables = args[:nt]
  idxs = args[nt:2 * nt]
  outs = args[2 * nt:3 * nt]
  idx_hv, rows_v, osem, gsem = args[3 * nt:]
  gsems = (gsem,)

  for f in range(nt):
    pltpu.sync_copy(idxs[f].at[pl.ds(base, BPW)], idx_hv)
    table = tables[f]

    @pl.loop(0, BPW, step=16)
    def _rows(i):
      v = idx_hv[pl.ds(i, 16)]
      for j in range(16):
        pltpu.async_copy(
            table.at[pl.ds(v[j], 1)], rows_v.at[pl.ds(i + j, 1)], gsems[0])

    pltpu.make_async_copy(table.at[pl.ds(0, BPW)], rows_v, gsems[0]).wait()
    pltpu.sync_copy(rows_v, outs[f].at[pl.ds(base, BPW)])


def _sc_gather(tables, idxs):
  nt = len(tables)
  mesh = plsc.VectorSubcoreMesh(core_axis_name="c", subcore_axis_name="s")
  out_t = tuple(
      jax.ShapeDtypeStruct((B, TD), jnp.float32) for _ in range(nt))
  if nt == 1:
    out_t = out_t[0]
  fn = pl.kernel(
      _gather_kernel,
      out_type=out_t,
      mesh=mesh,
      scratch_types=(
          [pltpu.VMEM((BPW,), jnp.int32),
           pltpu.VMEM((BPW, TD), jnp.float32)]
          + [pltpu.SemaphoreType.DMA for _ in range(2)]),
  )
  return fn(*tables, *idxs)


def _mlp_kernel(ea, eb, ec, ed, eq, num, vec,
                nw1, nb1, nw2, nb2, vw, vb, mw1, mb1, mw2, mb2,
                out):
  h = jnp.maximum(
      jnp.dot(num[...], nw1[...], preferred_element_type=jnp.float32)
      + nb1[...], 0.0)
  h = jnp.dot(h, nw2[...], preferred_element_type=jnp.float32) + nb2[...]
  v = jnp.dot(vec[...], vw[...], preferred_element_type=jnp.float32) + vb[...]
  feat = jnp.concatenate(
      [ea[...], eb[...], ec[...], ed[...], eq[...], h, v], axis=1)
  g = jnp.maximum(
      jnp.dot(feat, mw1[...], preferred_element_type=jnp.float32) + mb1[...],
      0.0)
  out[...] = (
      jnp.dot(g, mw2[...], preferred_element_type=jnp.float32) + mb2[...])


def _tc_mlp(emb_a, emb_b, emb_c, emb_d, emb_q, numericals, vec_emb,
            num_W1, num_b1, num_W2, num_b2, vec_W, vec_b,
            merge_W1, merge_b1, merge_W2, merge_b2):
  BB = 2048
  grid = (B // BB,)

  def batch_spec(width):
    return pl.BlockSpec((BB, width), lambda i: (i, 0))

  def full_spec(shape):
    return pl.BlockSpec(shape, lambda i: tuple(0 for _ in shape))

  return pl.pallas_call(
      _mlp_kernel,
      grid=grid,
      in_specs=[
          batch_spec(TD), batch_spec(TD), batch_spec(TD), batch_spec(TD),
          batch_spec(TD), batch_spec(NNUM), batch_spec(VD),
          full_spec(num_W1.shape), full_spec(num_b1.shape),
          full_spec(num_W2.shape), full_spec(num_b2.shape),
          full_spec(vec_W.shape), full_spec(vec_b.shape),
          full_spec(merge_W1.shape), full_spec(merge_b1.shape),
          full_spec(merge_W2.shape), full_spec(merge_b2.shape),
      ],
      out_specs=batch_spec(QED),
      out_shape=jax.ShapeDtypeStruct((B, QED), jnp.float32),
  )(emb_a, emb_b, emb_c, emb_d, emb_q, numericals, vec_emb,
    num_W1, num_b1, num_W2, num_b2, vec_W, vec_b,
    merge_W1, merge_b1, merge_W2, merge_b2)


def kernel(query_id, cat_a, cat_b, cat_c, cat_d, numericals, vec_emb,
           query_table, ct_a, ct_b, ct_c, ct_d,
           num_W1, num_b1, num_W2, num_b2,
           vec_W, vec_b,
           merge_W1, merge_b1, merge_W2, merge_b2):
  qid = query_id.astype(jnp.int32)
  ca = cat_a.astype(jnp.int32)
  cb = cat_b.astype(jnp.int32)
  cc = cat_c.astype(jnp.int32)
  cd = cat_d.astype(jnp.int32)

  ea, eb, ec, ed = _sc_gather((ct_a, ct_b, ct_c, ct_d), (ca, cb, cc, cd))
  eq = _sc_gather((query_table,), (qid,))

  return _tc_mlp(
      ea, eb, ec, ed, eq, numericals, vec_emb,
      num_W1, num_b1.reshape(1, -1), num_W2, num_b2.reshape(1, -1),
      vec_W, vec_b.reshape(1, -1),
      merge_W1, merge_b1.reshape(1, -1), merge_W2, merge_b2.reshape(1, -1))
